# Initial kernel scaffold; baseline (speedup 1.0000x reference)
import jax
import jax.numpy as jnp
from jax import lax
from jax.experimental import pallas as pl
from jax.experimental.pallas import tpu as pltpu
from jax.experimental.pallas import tpu_sc as plsc

_B, _S, _F = 1024, 512, 64
_NC, _NS = 2, 16
_NW = _NC * _NS
_ROWS_PER_W = _B // _NW
_CHUNK = 128
_NCHUNK = _S // _CHUNK
_LANES = 16


def _embed_body(tokens_hbm, table_hbm, pos_hbm, out_hbm,
                idx_v, rows_v, pos_v, sem):
    wid = lax.axis_index("s") * _NC + lax.axis_index("c")
    base = wid * _ROWS_PER_W

    pltpu.sync_copy(pos_hbm, pos_v)

    for j in range(_ROWS_PER_W):
        row = base + j
        pltpu.sync_copy(tokens_hbm.at[row], idx_v)
        copies = [
            pltpu.async_copy(
                table_hbm.at[idx_v.at[t]],
                rows_v.at[pl.ds(t * _CHUNK, _CHUNK)],
                sem,
            )
            for t in range(_NCHUNK)
        ]
        for c in copies:
            c.wait()

        @pl.loop(0, _S, unroll=8)
        def _add_pos(r):
            for k in range(_F // _LANES):
                sl = pl.ds(k * _LANES, _LANES)
                plsc.addupdate(rows_v.at[r, sl], pos_v[r, sl])

        pltpu.sync_copy(rows_v, out_hbm.at[row])


def kernel(tokens, node_span_starts, node_span_ends, embed_table, pos_emb):
    del node_span_starts, node_span_ends
    tokens3 = tokens.reshape(_B, _NCHUNK, _CHUNK)
    pos2 = pos_emb.reshape(_S, _F)
    mesh = plsc.VectorSubcoreMesh(core_axis_name="c", subcore_axis_name="s")
    out = pl.kernel(
        _embed_body,
        out_type=jax.ShapeDtypeStruct((_B, _S, _F), jnp.float32),
        mesh=mesh,
        scratch_types=[
            pltpu.VMEM((_NCHUNK, _CHUNK), jnp.int32),
            pltpu.VMEM((_S, _F), jnp.float32),
            pltpu.VMEM((_S, _F), jnp.float32),
            pltpu.SemaphoreType.DMA,
        ],
    )(tokens3, embed_table, pos2)
    return out


# speedup vs baseline: 3.3188x; 3.3188x over previous
import jax
import jax.numpy as jnp
from jax import lax
from jax.experimental import pallas as pl
from jax.experimental.pallas import tpu as pltpu
from jax.experimental.pallas import tpu_sc as plsc

_B, _S, _F = 1024, 512, 64
_NC, _NS = 2, 16
_NW = _NC * _NS
_ROWS_PER_W = _B // _NW
_CHUNK = 128
_NCHUNK = _S // _CHUNK
_LANES = 16


def _embed_body(tokens_hbm, table_hbm, pos_hbm, out_hbm,
                idx_v, rows_v, pos_v, sem):
    wid = lax.axis_index("s") * _NC + lax.axis_index("c")
    base = wid * _ROWS_PER_W

    pltpu.sync_copy(pos_hbm, pos_v)

    for j in range(_ROWS_PER_W):
        row = base + j
        pltpu.sync_copy(tokens_hbm.at[row], idx_v)
        copies = [
            pltpu.async_copy(
                table_hbm.at[idx_v.at[t]],
                rows_v.at[pl.ds(t * _CHUNK, _CHUNK)],
                sem,
            )
            for t in range(_NCHUNK)
        ]
        for c in copies:
            c.wait()

        @pl.loop(0, _S, unroll=8)
        def _add_pos(r):
            for k in range(_F // _LANES):
                sl = pl.ds(k * _LANES, _LANES)
                plsc.addupdate(rows_v.at[r, sl], pos_v[r, sl])

        pltpu.sync_copy(rows_v, out_hbm.at[row])


def kernel(tokens, node_span_starts, node_span_ends, embed_table, pos_emb):
    del node_span_starts, node_span_ends
    tokens3 = tokens.reshape(_B, _NCHUNK, _CHUNK)
    pos2 = pos_emb.reshape(_S, _F)
    mesh = plsc.VectorSubcoreMesh(core_axis_name="c", subcore_axis_name="s")
    out = pl.kernel(
        _embed_body,
        out_type=jax.ShapeDtypeStruct((_B, _S, _F), jnp.float32),
        mesh=mesh,
        compiler_params=pltpu.CompilerParams(use_tc_tiling_on_sc=False),
        scratch_types=[
            pltpu.VMEM((_NCHUNK, _CHUNK), jnp.int32),
            pltpu.VMEM((_S, _F), jnp.float32),
            pltpu.VMEM((_S, _F), jnp.float32),
            pltpu.SemaphoreType.DMA,
        ],
    )(tokens3, embed_table, pos2)
    return out


# speedup vs baseline: 3.7102x; 1.1179x over previous
import jax
import jax.numpy as jnp
from jax import lax
from jax.experimental import pallas as pl
from jax.experimental.pallas import tpu as pltpu
from jax.experimental.pallas import tpu_sc as plsc

_B, _S, _F = 1024, 512, 64
_NC, _NS = 2, 16
_NW = _NC * _NS
_ROWS_PER_W = _B // _NW
_CHUNK = 128
_NCHUNK = _S // _CHUNK
_LANES = 16


def _embed_body(tokens_hbm, table_hbm, pos_hbm, out_hbm,
                idx_v, rows_v, pos_v, gsem, ssem):
    wid = lax.axis_index("s") * _NC + lax.axis_index("c")
    base = wid * _ROWS_PER_W

    pltpu.sync_copy(pos_hbm, pos_v)

    def start_gather(j, b):
        pltpu.sync_copy(tokens_hbm.at[base + j], idx_v.at[b])
        return [
            pltpu.async_copy(
                table_hbm.at[idx_v.at[b, t]],
                rows_v.at[b, pl.ds(t * _CHUNK, _CHUNK)],
                gsem.at[b],
            )
            for t in range(_NCHUNK)
        ]

    g_desc = {0: start_gather(0, 0)}
    s_desc = {}
    for j in range(_ROWS_PER_W):
        b = j % 2
        if j + 1 < _ROWS_PER_W:
            if (j - 1) in s_desc:
                s_desc.pop(j - 1).wait()
            g_desc[j + 1] = start_gather(j + 1, 1 - b)
        for c in g_desc.pop(j):
            c.wait()

        @pl.loop(0, _S, unroll=8)
        def _add_pos(r):
            for k in range(_F // _LANES):
                sl = pl.ds(k * _LANES, _LANES)
                plsc.addupdate(rows_v.at[b, r, sl], pos_v[r, sl])

        s_desc[j] = pltpu.async_copy(rows_v.at[b], out_hbm.at[base + j],
                                     ssem.at[b])
    for d in s_desc.values():
        d.wait()


def kernel(tokens, node_span_starts, node_span_ends, embed_table, pos_emb):
    del node_span_starts, node_span_ends
    tokens3 = tokens.reshape(_B, _NCHUNK, _CHUNK)
    pos2 = pos_emb.reshape(_S, _F)
    mesh = plsc.VectorSubcoreMesh(core_axis_name="c", subcore_axis_name="s")
    out = pl.kernel(
        _embed_body,
        out_type=jax.ShapeDtypeStruct((_B, _S, _F), jnp.float32),
        mesh=mesh,
        compiler_params=pltpu.CompilerParams(use_tc_tiling_on_sc=False),
        scratch_types=[
            pltpu.VMEM((2, _NCHUNK, _CHUNK), jnp.int32),
            pltpu.VMEM((2, _S, _F), jnp.float32),
            pltpu.VMEM((_S, _F), jnp.float32),
            pltpu.SemaphoreType.DMA((2,)),
            pltpu.SemaphoreType.DMA((2,)),
        ],
    )(tokens3, embed_table, pos2)
    return out
